# baseline (speedup 1.0000x reference)
import numpy as np
import jax
import jax.numpy as jnp
from jax.experimental import pallas as pl
from jax.experimental.pallas import tpu as pltpu

_NG = 4
_GB = 64
_W = _GB * 128


def _build_perm_matrices():
    P = np.zeros((2, 2, 128, 128), dtype=np.float32)
    for v in (0, 1):
        PA = np.zeros((128, 128), dtype=np.float32)
        PB = np.zeros((128, 128), dtype=np.float32)
        for l in range(128):
            s = (l ^ (l >> 1)) ^ (64 * v)
            if l % 2 == 0:
                PA[s, l] = 1.0
            else:
                PB[s, l] = 1.0
        P[v, 0] = 0.5 * (PA + PB)
        P[v, 1] = 0.5 * (PA - PB)
    return P


_P_NP = _build_perm_matrices()

_BASES = []
for _i in range(_NG):
    _g9 = (_GB * _i) ^ ((_GB * _i) >> 1)
    _BASES.append(_g9 & ~(_GB - 1))


def _in_copies(phi_ref, abuf_ref, bbuf_ref, insem, k):
    baseA = _BASES[k]
    baseB = baseA ^ 384
    copies = []
    for h in (0, 1):
        rows = pl.ds(h * 64, 64)
        copies.append(pltpu.make_async_copy(
            phi_ref.at[rows, pl.ds(baseA * 128, _W)],
            abuf_ref.at[k % 2, rows], insem.at[k % 2, 0, h]))
        copies.append(pltpu.make_async_copy(
            phi_ref.at[rows, pl.ds(baseB * 128, _W)],
            bbuf_ref.at[k % 2, rows], insem.at[k % 2, 1, h]))
    return copies


def _out_copies(out_ref, obuf_ref, outsem, k):
    copies = []
    for h in (0, 1):
        rows = pl.ds(h * 64, 64)
        copies.append(pltpu.make_async_copy(
            obuf_ref.at[k % 2, rows, pl.ds(0, _W)],
            out_ref.at[rows, pl.ds(k * _W, _W)], outsem.at[k % 2, 0, h]))
        copies.append(pltpu.make_async_copy(
            obuf_ref.at[k % 2, rows, pl.ds(_W, _W)],
            out_ref.at[rows, pl.ds(32768 + k * _W, _W)],
            outsem.at[k % 2, 1, h]))
    return copies


def _body(p_ref, phi_ref, out_ref, abuf_ref, bbuf_ref, obuf_ref,
          insem, outsem):
    pmats = ((p_ref[0, 0], p_ref[0, 1]), (p_ref[1, 0], p_ref[1, 1]))

    for c in _in_copies(phi_ref, abuf_ref, bbuf_ref, insem, 0):
        c.start()
    for c in _in_copies(phi_ref, abuf_ref, bbuf_ref, insem, 1):
        c.start()

    for k in range(_NG):
        for c in _in_copies(phi_ref, abuf_ref, bbuf_ref, insem, k):
            c.wait()
        if k >= 2:
            for c in _out_copies(out_ref, obuf_ref, outsem, k - 2):
                c.wait()
        off = 32 * (k & 1)
        for t in range(_GB):
            p = ((t ^ (t >> 1)) ^ off) * 128
            a = abuf_ref[k % 2, :, p:p + 128]
            b = bbuf_ref[k % 2, :, p:p + 128]
            pp, pm = pmats[t & 1]
            u = jax.lax.dot_general(
                a + b, pp, (((1,), (0,)), ((), ())),
                preferred_element_type=jnp.float32)
            w = jax.lax.dot_general(
                a - b, pm, (((1,), (0,)), ((), ())),
                preferred_element_type=jnp.float32)
            obuf_ref[k % 2, :, t * 128:(t + 1) * 128] = u + w
            obuf_ref[k % 2, :, _W + t * 128:_W + (t + 1) * 128] = u - w
        for c in _out_copies(out_ref, obuf_ref, outsem, k):
            c.start()
        if k + 2 < _NG:
            for c in _in_copies(phi_ref, abuf_ref, bbuf_ref, insem, k + 2):
                c.start()

    for k in (_NG - 2, _NG - 1):
        for c in _out_copies(out_ref, obuf_ref, outsem, k):
            c.wait()


def kernel(phi):
    p_mat = jnp.asarray(_P_NP)
    return pl.pallas_call(
        _body,
        in_specs=[
            pl.BlockSpec(memory_space=pltpu.MemorySpace.VMEM),
            pl.BlockSpec(memory_space=pltpu.MemorySpace.HBM),
        ],
        out_specs=pl.BlockSpec(memory_space=pltpu.MemorySpace.HBM),
        out_shape=jax.ShapeDtypeStruct((128, 65536), jnp.float32),
        scratch_shapes=[
            pltpu.VMEM((2, 128, _W), jnp.float32),
            pltpu.VMEM((2, 128, _W), jnp.float32),
            pltpu.VMEM((2, 128, 2 * _W), jnp.float32),
            pltpu.SemaphoreType.DMA((2, 2, 2)),
            pltpu.SemaphoreType.DMA((2, 2, 2)),
        ],
    )(p_mat, phi)


# speedup vs baseline: 1.0077x; 1.0077x over previous
import numpy as np
import jax
import jax.numpy as jnp
from jax.experimental import pallas as pl
from jax.experimental.pallas import tpu as pltpu

_NG = 4
_GB = 64
_W = _GB * 128


def _build_perm_matrices():
    P = np.zeros((2, 2, 128, 128), dtype=np.float32)
    for v in (0, 1):
        PA = np.zeros((128, 128), dtype=np.float32)
        PB = np.zeros((128, 128), dtype=np.float32)
        for l in range(128):
            s = (l ^ (l >> 1)) ^ (64 * v)
            if l % 2 == 0:
                PA[s, l] = 1.0
            else:
                PB[s, l] = 1.0
        P[v, 0] = 0.5 * (PA + PB)
        P[v, 1] = 0.5 * (PA - PB)
    return P


_P_NP = _build_perm_matrices()

_BASES = []
for _i in range(_NG):
    _g9 = (_GB * _i) ^ ((_GB * _i) >> 1)
    _BASES.append(_g9 & ~(_GB - 1))


def _in_copies(phi_ref, abuf_ref, bbuf_ref, insem, k):
    baseA = _BASES[k]
    baseB = baseA ^ 384
    return (
        pltpu.make_async_copy(
            phi_ref.at[:, pl.ds(baseA * 128, _W)],
            abuf_ref.at[k % 2], insem.at[k % 2, 0]),
        pltpu.make_async_copy(
            phi_ref.at[:, pl.ds(baseB * 128, _W)],
            bbuf_ref.at[k % 2], insem.at[k % 2, 1]),
    )


def _out_copies(out_ref, obuf_ref, outsem, k):
    return (
        pltpu.make_async_copy(
            obuf_ref.at[k % 2, :, pl.ds(0, _W)],
            out_ref.at[:, pl.ds(k * _W, _W)], outsem.at[k % 2, 0]),
        pltpu.make_async_copy(
            obuf_ref.at[k % 2, :, pl.ds(_W, _W)],
            out_ref.at[:, pl.ds(32768 + k * _W, _W)], outsem.at[k % 2, 1]),
    )


def _body(p_ref, phi_ref, out_ref, abuf_ref, bbuf_ref, obuf_ref,
          insem, outsem):
    pmats = ((p_ref[0, 0], p_ref[0, 1]), (p_ref[1, 0], p_ref[1, 1]))

    for c in _in_copies(phi_ref, abuf_ref, bbuf_ref, insem, 0):
        c.start()
    for c in _in_copies(phi_ref, abuf_ref, bbuf_ref, insem, 1):
        c.start()

    for k in range(_NG):
        for c in _in_copies(phi_ref, abuf_ref, bbuf_ref, insem, k):
            c.wait()
        if k >= 2:
            for c in _out_copies(out_ref, obuf_ref, outsem, k - 2):
                c.wait()
        off = 32 * (k & 1)
        for t in range(_GB):
            p = ((t ^ (t >> 1)) ^ off) * 128
            a = abuf_ref[k % 2, :, p:p + 128]
            b = bbuf_ref[k % 2, :, p:p + 128]
            pp, pm = pmats[t & 1]
            u = jax.lax.dot_general(
                a + b, pp, (((1,), (0,)), ((), ())),
                preferred_element_type=jnp.float32)
            w = jax.lax.dot_general(
                a - b, pm, (((1,), (0,)), ((), ())),
                preferred_element_type=jnp.float32)
            obuf_ref[k % 2, :, t * 128:(t + 1) * 128] = u + w
            obuf_ref[k % 2, :, _W + t * 128:_W + (t + 1) * 128] = u - w
        for c in _out_copies(out_ref, obuf_ref, outsem, k):
            c.start()
        if k + 2 < _NG:
            for c in _in_copies(phi_ref, abuf_ref, bbuf_ref, insem, k + 2):
                c.start()

    for k in (_NG - 2, _NG - 1):
        for c in _out_copies(out_ref, obuf_ref, outsem, k):
            c.wait()


def kernel(phi):
    p_mat = jnp.asarray(_P_NP)
    return pl.pallas_call(
        _body,
        in_specs=[
            pl.BlockSpec(memory_space=pltpu.MemorySpace.VMEM),
            pl.BlockSpec(memory_space=pltpu.MemorySpace.HBM),
        ],
        out_specs=pl.BlockSpec(memory_space=pltpu.MemorySpace.HBM),
        out_shape=jax.ShapeDtypeStruct((128, 65536), jnp.float32),
        scratch_shapes=[
            pltpu.VMEM((2, 128, _W), jnp.float32),
            pltpu.VMEM((2, 128, _W), jnp.float32),
            pltpu.VMEM((2, 128, 2 * _W), jnp.float32),
            pltpu.SemaphoreType.DMA((2, 2)),
            pltpu.SemaphoreType.DMA((2, 2)),
        ],
    )(p_mat, phi)
